# Initial kernel scaffold; baseline (speedup 1.0000x reference)
import jax
import jax.numpy as jnp
from jax.experimental import pallas as pl


def kernel(x, edge_index, W1, b1, W2, b2):
    raise NotImplementedError("write your pallas kernel here")


# speedup vs baseline: 9.8634x; 9.8634x over previous
import functools

import jax
import jax.numpy as jnp
from jax import lax
from jax.experimental import pallas as pl
from jax.experimental.pallas import tpu as pltpu
from jax.experimental.pallas import tpu_sc as plsc

N_REAL = 10000
N_PAD = 10240
DUMMY = 10000
D = 128
E_REAL = 320000
NW = 32
CHUNK = 128
ROWS_W = 80
E_PAD = NW * ROWS_W * CHUNK
SLICE = N_PAD // 16
BLK = 1024
GRID = N_PAD // BLK

_MESH = dict(core_axis_name="c", subcore_axis_name="s")


def _deg_partials(dst2):
    @functools.partial(
        pl.kernel,
        out_type=jax.ShapeDtypeStruct((2, N_PAD), jnp.float32),
        mesh=plsc.VectorSubcoreMesh(**_MESH),
        scratch_types=[
            pltpu.VMEM((ROWS_W, CHUNK), jnp.int32),
            pltpu.VMEM((CHUNK,), jnp.float32),
            pltpu.VMEM((SLICE,), jnp.float32),
            pltpu.VMEM_SHARED((N_PAD,), jnp.float32),
        ],
    )
    def k(dst_hbm, out_hbm, dst_v, ones_v, zbuf, cnt):
        c = lax.axis_index("c")
        s = lax.axis_index("s")
        wid = s * 2 + c

        def fill_ones(i, _):
            ones_v[pl.ds(i * 16, 16)] = jnp.ones((16,), jnp.float32)
            return 0

        lax.fori_loop(0, CHUNK // 16, fill_ones, 0)

        def fill_zeros(i, _):
            zbuf[pl.ds(i * 16, 16)] = jnp.zeros((16,), jnp.float32)
            return 0

        lax.fori_loop(0, SLICE // 16, fill_zeros, 0)
        pltpu.sync_copy(zbuf, cnt.at[pl.ds(s * SLICE, SLICE)])
        plsc.subcore_barrier()

        pltpu.sync_copy(dst_hbm.at[pl.ds(wid * ROWS_W, ROWS_W)], dst_v)

        def body(j, _):
            pltpu.sync_copy(ones_v, cnt.at[dst_v.at[j]], add=True)
            return 0

        lax.fori_loop(0, ROWS_W, body, 0)
        plsc.subcore_barrier()
        pltpu.sync_copy(cnt.at[pl.ds(s * SLICE, SLICE)],
                        out_hbm.at[c, pl.ds(s * SLICE, SLICE)])

    return k(dst2)


def _spmm_partials(g, src2, dst2):
    @functools.partial(
        pl.kernel,
        out_type=jax.ShapeDtypeStruct((2, N_PAD, D), jnp.float32),
        mesh=plsc.VectorSubcoreMesh(**_MESH),
        scratch_types=[
            pltpu.VMEM((ROWS_W, CHUNK), jnp.int32),
            pltpu.VMEM((ROWS_W, CHUNK), jnp.int32),
            pltpu.VMEM((CHUNK, D), jnp.float32),
            pltpu.VMEM_SHARED((N_PAD, D), jnp.float32),
            pltpu.SemaphoreType.DMA,
        ],
    )
    def k(g_hbm, src_hbm, dst_hbm, out_hbm, src_v, dst_v, buf, acc, sem):
        c = lax.axis_index("c")
        s = lax.axis_index("s")
        wid = s * 2 + c

        def fill_zeros(i, _):
            r = i // (D // 16)
            col = (i % (D // 16)) * 16
            buf[r, pl.ds(col, 16)] = jnp.zeros((16,), jnp.float32)
            return 0

        lax.fori_loop(0, CHUNK * (D // 16), fill_zeros, 0)
        for j in range(SLICE // CHUNK):
            pltpu.sync_copy(buf, acc.at[pl.ds(s * SLICE + j * CHUNK, CHUNK)])
        plsc.subcore_barrier()

        pltpu.sync_copy(src_hbm.at[pl.ds(wid * ROWS_W, ROWS_W)], src_v)
        pltpu.sync_copy(dst_hbm.at[pl.ds(wid * ROWS_W, ROWS_W)], dst_v)

        def body(j, _):
            pltpu.async_copy(g_hbm.at[src_v.at[j]], buf, sem).wait()
            pltpu.sync_copy(buf, acc.at[dst_v.at[j]], add=True)
            return 0

        lax.fori_loop(0, ROWS_W, body, 0)
        plsc.subcore_barrier()
        for j in range(SLICE // CHUNK):
            pltpu.sync_copy(acc.at[pl.ds(s * SLICE + j * CHUNK, CHUNK)],
                            out_hbm.at[c, pl.ds(s * SLICE + j * CHUNK, CHUNK)])

    return k(g, src2, dst2)


def _tc1(degp, x_pad, W1):
    def body(deg_ref, x_ref, w_ref, g_ref, dinv_ref):
        i = pl.program_id(0)
        deg = deg_ref[0, pl.ds(i * BLK, BLK)] + deg_ref[1, pl.ds(i * BLK, BLK)] + 1.0
        dinv = lax.rsqrt(deg)
        dinv_ref[...] = dinv[:, None]
        g_ref[...] = dinv[:, None] * jnp.dot(
            x_ref[...], w_ref[...], preferred_element_type=jnp.float32)

    return pl.pallas_call(
        body,
        grid=(GRID,),
        in_specs=[
            pl.BlockSpec((2, N_PAD), lambda i: (0, 0)),
            pl.BlockSpec((BLK, D), lambda i: (i, 0)),
            pl.BlockSpec((D, D), lambda i: (0, 0)),
        ],
        out_specs=[
            pl.BlockSpec((BLK, D), lambda i: (i, 0)),
            pl.BlockSpec((BLK, 1), lambda i: (i, 0)),
        ],
        out_shape=[
            jax.ShapeDtypeStruct((N_PAD, D), jnp.float32),
            jax.ShapeDtypeStruct((N_PAD, 1), jnp.float32),
        ],
    )(degp, x_pad, W1)


def _tc2(P, g1, dinv, b1, W2):
    def body(p_ref, g_ref, dinv_ref, b_ref, w_ref, o_ref):
        dinv_c = dinv_ref[...]
        h = jnp.maximum(dinv_c * (p_ref[0] + p_ref[1] + g_ref[...]) + b_ref[...], 0.0)
        o_ref[...] = dinv_c * jnp.dot(h, w_ref[...], preferred_element_type=jnp.float32)

    return pl.pallas_call(
        body,
        grid=(GRID,),
        in_specs=[
            pl.BlockSpec((2, BLK, D), lambda i: (0, i, 0)),
            pl.BlockSpec((BLK, D), lambda i: (i, 0)),
            pl.BlockSpec((BLK, 1), lambda i: (i, 0)),
            pl.BlockSpec((1, D), lambda i: (0, 0)),
            pl.BlockSpec((D, D), lambda i: (0, 0)),
        ],
        out_specs=pl.BlockSpec((BLK, D), lambda i: (i, 0)),
        out_shape=jax.ShapeDtypeStruct((N_PAD, D), jnp.float32),
    )(P, g1, dinv, b1, W2)


def _tc3(P, g2, dinv, b2):
    def body(p_ref, g_ref, dinv_ref, b_ref, o_ref):
        o_ref[...] = dinv_ref[...] * (p_ref[0] + p_ref[1] + g_ref[...]) + b_ref[...]

    return pl.pallas_call(
        body,
        grid=(GRID,),
        in_specs=[
            pl.BlockSpec((2, BLK, D), lambda i: (0, i, 0)),
            pl.BlockSpec((BLK, D), lambda i: (i, 0)),
            pl.BlockSpec((BLK, 1), lambda i: (i, 0)),
            pl.BlockSpec((1, D), lambda i: (0, 0)),
        ],
        out_specs=pl.BlockSpec((BLK, D), lambda i: (i, 0)),
        out_shape=jax.ShapeDtypeStruct((N_PAD, D), jnp.float32),
    )(P, g2, dinv, b2)


def kernel(x, edge_index, W1, b1, W2, b2):
    src = edge_index[0].astype(jnp.int32)
    dst = edge_index[1].astype(jnp.int32)
    pad = jnp.full((E_PAD - E_REAL,), DUMMY, jnp.int32)
    src2 = jnp.concatenate([src, pad]).reshape(NW * ROWS_W, CHUNK)
    dst2 = jnp.concatenate([dst, pad]).reshape(NW * ROWS_W, CHUNK)
    x_pad = jnp.zeros((N_PAD, D), jnp.float32).at[:N_REAL].set(x)

    degp = _deg_partials(dst2)
    g1, dinv = _tc1(degp, x_pad, W1)
    P1 = _spmm_partials(g1, src2, dst2)
    g2 = _tc2(P1, g1, dinv, b1.reshape(1, D), W2)
    P2 = _spmm_partials(g2, src2, dst2)
    z = _tc3(P2, g2, dinv, b2.reshape(1, D))
    return z[:N_REAL]
